# baseline (speedup 1.0000x reference)
import functools

import jax
import jax.numpy as jnp
from jax import lax
from jax.experimental import pallas as pl
from jax.experimental.pallas import tpu as pltpu
from jax.experimental.pallas import tpu_sc as plsc

BATCH = 16384
DIM = 32
NUM_CORES = 2
NUM_SUBCORES = 16
NUM_WORKERS = NUM_CORES * NUM_SUBCORES
ROWS_PER_WORKER = BATCH // NUM_WORKERS
CHUNK = 128
NUM_CHUNKS = ROWS_PER_WORKER // CHUNK
GROUPS_PER_CHUNK = CHUNK // 16

_MESH = plsc.VectorSubcoreMesh(
    core_axis_name="c", subcore_axis_name="s", num_cores=NUM_CORES
)


@functools.partial(
    pl.kernel,
    out_type=jax.ShapeDtypeStruct((BATCH,), jnp.float32),
    mesh=_MESH,
    compiler_params=pltpu.CompilerParams(
        needs_layout_passes=False, use_tc_tiling_on_sc=False
    ),
    scratch_types=[
        pltpu.VMEM((NUM_CHUNKS, CHUNK), jnp.int32),
        pltpu.VMEM((NUM_CHUNKS, CHUNK), jnp.int32),
        pltpu.VMEM((NUM_CHUNKS, CHUNK), jnp.int32),
        pltpu.VMEM((ROWS_PER_WORKER, DIM), jnp.int32),
        pltpu.VMEM((ROWS_PER_WORKER, DIM), jnp.int32),
        pltpu.VMEM((ROWS_PER_WORKER, DIM), jnp.int32),
        pltpu.VMEM((ROWS_PER_WORKER,), jnp.float32),
        pltpu.SemaphoreType.DMA,
        pltpu.SemaphoreType.DMA,
        pltpu.SemaphoreType.DMA,
        pltpu.SemaphoreType.DMA,
    ],
)
def _complex_score_sc(
    h_hbm,
    r_hbm,
    t_hbm,
    ent_hbm,
    rel_hbm,
    out_hbm,
    hv,
    rv,
    tv,
    hrow,
    rrow,
    trow,
    scores,
    *sems,
):
    wid = lax.axis_index("s") * NUM_CORES + lax.axis_index("c")
    base = wid * ROWS_PER_WORKER

    copies = []
    for k in range(NUM_CHUNKS):
        off = base + k * CHUNK
        pltpu.sync_copy(h_hbm.at[pl.ds(off, CHUNK)], hv.at[k])
        pltpu.sync_copy(r_hbm.at[pl.ds(off, CHUNK)], rv.at[k])
        pltpu.sync_copy(t_hbm.at[pl.ds(off, CHUNK)], tv.at[k])
        dst = pl.ds(k * CHUNK, CHUNK)
        copies.append(
            [
                pltpu.async_copy(ent_hbm.at[hv.at[k]], hrow.at[dst], sems[k]),
                pltpu.async_copy(rel_hbm.at[rv.at[k]], rrow.at[dst], sems[k]),
                pltpu.async_copy(ent_hbm.at[tv.at[k]], trow.at[dst], sems[k]),
            ]
        )

    lanes = lax.iota(jnp.int32, 16)

    for k in range(NUM_CHUNKS):
        for c in copies[k]:
            c.wait()

        def group_body(g, _, k=k):
            rows = k * CHUNK + g * 16 + lanes
            acc = jnp.zeros((16,), jnp.float32)
            for d in range(DIM):
                dv = jnp.full((16,), d, jnp.int32)
                hw = plsc.load_gather(hrow, [rows, dv])
                rw = plsc.load_gather(rrow, [rows, dv])
                tw = plsc.load_gather(trow, [rows, dv])
                a, b = plsc.unpack(
                    plsc.bitcast(hw, jnp.bfloat16),
                    format=plsc.PackFormat.INTERLEAVED,
                )
                cr, ci = plsc.unpack(
                    plsc.bitcast(rw, jnp.bfloat16),
                    format=plsc.PackFormat.INTERLEAVED,
                )
                e, f = plsc.unpack(
                    plsc.bitcast(tw, jnp.bfloat16),
                    format=plsc.PackFormat.INTERLEAVED,
                )
                acc = acc + cr * (a * e + b * f) + ci * (a * f - b * e)
            scores[pl.ds(k * CHUNK + g * 16, 16)] = acc
            return 0

        lax.fori_loop(0, GROUPS_PER_CHUNK, group_body, 0)

    pltpu.sync_copy(scores, out_hbm.at[pl.ds(base, ROWS_PER_WORKER)])


_PACK_COLS = 4096
_NUM_ROWS = 1_000_000


def _pack_t_body(re_ref, im_ref, out_ref):
    eye = jnp.eye(DIM, dtype=jnp.float32)
    dims = (((0,), (0,)), ((), ()))
    re_t = lax.dot_general(re_ref[...], eye, dims)
    im_t = lax.dot_general(im_ref[...], eye, dims)
    rbits = lax.bitcast_convert_type(re_t, jnp.uint32)
    ibits = lax.bitcast_convert_type(im_t, jnp.uint32)
    word = (rbits >> 16) | (ibits & jnp.uint32(0xFFFF0000))
    out_ref[...] = lax.bitcast_convert_type(word, jnp.int32)


def _pack_t(real_t, imag_t):
    grid = (_NUM_ROWS + _PACK_COLS - 1) // _PACK_COLS
    return pl.pallas_call(
        _pack_t_body,
        grid=(grid,),
        in_specs=[
            pl.BlockSpec((DIM, _PACK_COLS), lambda j: (0, j)),
            pl.BlockSpec((DIM, _PACK_COLS), lambda j: (0, j)),
        ],
        out_specs=pl.BlockSpec((_PACK_COLS, DIM), lambda j: (j, 0)),
        out_shape=jax.ShapeDtypeStruct((_NUM_ROWS, DIM), jnp.int32),
    )(real_t, imag_t)


def kernel(triples, ent_real, ent_imag, rel_real, rel_imag):
    h = jnp.asarray(triples[:, 0], jnp.int32)
    r = jnp.asarray(triples[:, 1], jnp.int32)
    t = jnp.asarray(triples[:, 2], jnp.int32)
    ent = _pack_t(ent_real.T, ent_imag.T)
    rel = _pack_t(rel_real.T, rel_imag.T)
    return _complex_score_sc(h, r, t, ent, rel)


# speedup vs baseline: 1.0922x; 1.0922x over previous
import functools

import jax
import jax.numpy as jnp
from jax import lax
from jax.experimental import pallas as pl
from jax.experimental.pallas import tpu as pltpu
from jax.experimental.pallas import tpu_sc as plsc

BATCH = 16384
DIM = 32
NUM_CORES = 2
NUM_SUBCORES = 16
NUM_WORKERS = NUM_CORES * NUM_SUBCORES
ROWS_PER_WORKER = BATCH // NUM_WORKERS
CHUNK = 128
NUM_CHUNKS = ROWS_PER_WORKER // CHUNK
GROUPS_PER_CHUNK = CHUNK // 16

_MESH = plsc.VectorSubcoreMesh(
    core_axis_name="c", subcore_axis_name="s", num_cores=NUM_CORES
)


@functools.partial(
    pl.kernel,
    out_type=jax.ShapeDtypeStruct((BATCH,), jnp.float32),
    mesh=_MESH,
    compiler_params=pltpu.CompilerParams(
        needs_layout_passes=False, use_tc_tiling_on_sc=False
    ),
    scratch_types=[
        pltpu.VMEM((NUM_CHUNKS, CHUNK), jnp.int32),
        pltpu.VMEM((NUM_CHUNKS, CHUNK), jnp.int32),
        pltpu.VMEM((NUM_CHUNKS, CHUNK), jnp.int32),
        pltpu.VMEM((ROWS_PER_WORKER, DIM), jnp.int32),
        pltpu.VMEM((ROWS_PER_WORKER, DIM), jnp.int32),
        pltpu.VMEM((ROWS_PER_WORKER, DIM), jnp.int32),
        pltpu.VMEM((ROWS_PER_WORKER,), jnp.float32),
        pltpu.SemaphoreType.DMA,
        pltpu.SemaphoreType.DMA,
        pltpu.SemaphoreType.DMA,
        pltpu.SemaphoreType.DMA,
    ],
)
def _complex_score_sc(
    h_hbm,
    r_hbm,
    t_hbm,
    ent_hbm,
    rel_hbm,
    out_hbm,
    hv,
    rv,
    tv,
    hrow,
    rrow,
    trow,
    scores,
    *sems,
):
    wid = lax.axis_index("s") * NUM_CORES + lax.axis_index("c")
    base = wid * ROWS_PER_WORKER

    copies = []
    for k in range(NUM_CHUNKS):
        off = base + k * CHUNK
        pltpu.sync_copy(h_hbm.at[pl.ds(off, CHUNK)], hv.at[k])
        pltpu.sync_copy(r_hbm.at[pl.ds(off, CHUNK)], rv.at[k])
        pltpu.sync_copy(t_hbm.at[pl.ds(off, CHUNK)], tv.at[k])
        dst = pl.ds(k * CHUNK, CHUNK)
        copies.append(
            [
                pltpu.async_copy(ent_hbm.at[hv.at[k]], hrow.at[dst], sems[k]),
                pltpu.async_copy(rel_hbm.at[rv.at[k]], rrow.at[dst], sems[k]),
                pltpu.async_copy(ent_hbm.at[tv.at[k]], trow.at[dst], sems[k]),
            ]
        )

    lanes = lax.iota(jnp.int32, 16)

    for k in range(NUM_CHUNKS):
        for c in copies[k]:
            c.wait()

        def group_body(g, _, k=k):
            rows = k * CHUNK + g * 16 + lanes
            acc = jnp.zeros((16,), jnp.float32)
            for d in range(DIM):
                dv = jnp.full((16,), d, jnp.int32)
                hw = plsc.load_gather(hrow, [rows, dv])
                rw = plsc.load_gather(rrow, [rows, dv])
                tw = plsc.load_gather(trow, [rows, dv])
                a, b = plsc.unpack(
                    plsc.bitcast(hw, jnp.bfloat16),
                    format=plsc.PackFormat.INTERLEAVED,
                )
                cr, ci = plsc.unpack(
                    plsc.bitcast(rw, jnp.bfloat16),
                    format=plsc.PackFormat.INTERLEAVED,
                )
                e, f = plsc.unpack(
                    plsc.bitcast(tw, jnp.bfloat16),
                    format=plsc.PackFormat.INTERLEAVED,
                )
                acc = acc + cr * (a * e + b * f) + ci * (a * f - b * e)
            scores[pl.ds(k * CHUNK + g * 16, 16)] = acc
            return 0

        lax.fori_loop(0, GROUPS_PER_CHUNK, group_body, 0)

    pltpu.sync_copy(scores, out_hbm.at[pl.ds(base, ROWS_PER_WORKER)])


_PACK_COLS = 4096
_NUM_ROWS = 1_000_000


def _pack_t_body(re_ref, im_ref, out_ref):
    rbits = lax.bitcast_convert_type(re_ref[...], jnp.uint32)
    ibits = lax.bitcast_convert_type(im_ref[...], jnp.uint32)
    word = (rbits >> 16) | (ibits & jnp.uint32(0xFFFF0000))
    out_ref[...] = lax.bitcast_convert_type(word.T, jnp.int32)


def _pack_t(real_t, imag_t):
    grid = (_NUM_ROWS + _PACK_COLS - 1) // _PACK_COLS
    return pl.pallas_call(
        _pack_t_body,
        grid=(grid,),
        in_specs=[
            pl.BlockSpec((DIM, _PACK_COLS), lambda j: (0, j)),
            pl.BlockSpec((DIM, _PACK_COLS), lambda j: (0, j)),
        ],
        out_specs=pl.BlockSpec((_PACK_COLS, DIM), lambda j: (j, 0)),
        out_shape=jax.ShapeDtypeStruct((_NUM_ROWS, DIM), jnp.int32),
    )(real_t, imag_t)


def kernel(triples, ent_real, ent_imag, rel_real, rel_imag):
    h = jnp.asarray(triples[:, 0], jnp.int32)
    r = jnp.asarray(triples[:, 1], jnp.int32)
    t = jnp.asarray(triples[:, 2], jnp.int32)
    ent = _pack_t(ent_real.T, ent_imag.T)
    rel = _pack_t(rel_real.T, rel_imag.T)
    return _complex_score_sc(h, r, t, ent, rel)
